# baseline (speedup 1.0000x reference)
import functools

import jax
import jax.numpy as jnp
from jax import lax
from jax.experimental import pallas as pl
from jax.experimental.pallas import tpu as pltpu
from jax.experimental.pallas import tpu_sc as plsc

NC = 2
NS = 16
NW = NC * NS
CHUNK = 128


def _make_agg(n_nodes, n_chunks, width):
    rows_per_tile = ((n_nodes + NS) + NS * 8 - 1) // (NS * 8) * 8
    acc_rows = rows_per_tile * NS

    mesh = plsc.VectorSubcoreMesh(
        core_axis_name="c", subcore_axis_name="s", num_cores=NC,
        num_subcores=NS)

    @functools.partial(
        pl.kernel,
        out_type=jax.ShapeDtypeStruct((NC, acc_rows, width), jnp.float32),
        mesh=mesh,
        scratch_types=[
            pltpu.VMEM((n_chunks, CHUNK), jnp.int32),
            pltpu.VMEM((n_chunks, CHUNK), jnp.int32),
            pltpu.VMEM((CHUNK, width), jnp.float32),
            pltpu.VMEM((CHUNK, width), jnp.float32),
            pltpu.VMEM_SHARED((acc_rows, width), jnp.float32),
            pltpu.VMEM_SHARED((acc_rows, width), jnp.float32),
            pltpu.SemaphoreType.DMA,
            pltpu.SemaphoreType.DMA,
        ],
        compiler_params=pltpu.CompilerParams(use_tc_tiling_on_sc=False),
    )
    def agg(src_hbm, dst_hbm, table_hbm, zeros_hbm, out_hbm,
            src_v, dst_v, rows_a, rows_b, acc_sh, table_sh, sem_a, sem_b):
        c = lax.axis_index("c")
        s = lax.axis_index("s")
        wid = c * NS + s

        z0 = s * rows_per_tile
        pltpu.sync_copy(zeros_hbm.at[pl.ds(z0, rows_per_tile)],
                        acc_sh.at[pl.ds(z0, rows_per_tile)])
        pltpu.sync_copy(table_hbm.at[pl.ds(z0, rows_per_tile)],
                        table_sh.at[pl.ds(z0, rows_per_tile)])
        pltpu.sync_copy(src_hbm.at[wid], src_v)
        pltpu.sync_copy(dst_hbm.at[wid], dst_v)
        plsc.subcore_barrier()

        def chunk_body(g, carry):
            ca = pltpu.async_copy(
                table_sh.at[src_v.at[2 * g]], rows_a, sem_a)
            cb = pltpu.async_copy(
                table_sh.at[src_v.at[2 * g + 1]], rows_b, sem_b)
            ca.wait()
            cb.wait()
            pltpu.sync_copy(rows_a, acc_sh.at[dst_v.at[2 * g]], add=True)
            pltpu.sync_copy(rows_b, acc_sh.at[dst_v.at[2 * g + 1]], add=True)
            return carry

        lax.fori_loop(0, n_chunks // 2, chunk_body, 0)
        plsc.subcore_barrier()

        pltpu.sync_copy(acc_sh.at[pl.ds(z0, rows_per_tile)],
                        out_hbm.at[c, pl.ds(z0, rows_per_tile)])

    return agg


def _make_dense(n_nodes, d, p, q, block_rows):
    grid = (n_nodes // block_rows,)
    w = q + 8

    def body(u_ref, it_ref, ps_ref, wf_ref, bf_ref, wg_ref, bg_ref,
             wz_ref, bz_ref, wa_ref, ba_ref, out_ref):
        x = u_ref[...]
        dn = (((1,), (1,)), ((), ()))
        fu = jax.nn.softplus(
            lax.dot_general(x, wf_ref[...], dn,
                            preferred_element_type=jnp.float32) + bf_ref[...])
        gu = jax.nn.softplus(
            lax.dot_general(x, wg_ref[...], dn,
                            preferred_element_type=jnp.float32) + bg_ref[...])
        zu = jnp.tanh(
            lax.dot_general(x, wz_ref[...], dn,
                            preferred_element_type=jnp.float32) + bz_ref[...])
        ps = ps_ref[0] + ps_ref[1]
        nbr_sum = ps[:, :q]
        deg = ps[:, q:q + 1]
        nbr_mean = jnp.where(deg > 0, nbr_sum / jnp.maximum(deg, 1.0), 0.0)
        agi = jnp.concatenate([it_ref[...], nbr_mean], axis=1)
        ag = jnp.maximum(
            lax.dot_general(agi, wa_ref[...], dn,
                            preferred_element_type=jnp.float32) + ba_ref[...],
            0.0)
        du = -fu * x + gu * jnp.concatenate([zu, ag], axis=1)
        up = x[:, :p]
        dup = du[:, :p]
        s1 = jnp.sum(dup * up, axis=1, keepdims=True)
        s2 = jnp.sum(up * up, axis=1, keepdims=True)
        dup = dup - (s1 / s2) * up
        out_ref[...] = jnp.concatenate([dup, du[:, p:]], axis=1)

    return pl.pallas_call(
        body,
        grid=grid,
        in_specs=[
            pl.BlockSpec((block_rows, d), lambda i: (i, 0)),
            pl.BlockSpec((block_rows, q), lambda i: (i, 0)),
            pl.BlockSpec((NC, block_rows, w), lambda i: (0, i, 0)),
            pl.BlockSpec((d, d), lambda i: (0, 0)),
            pl.BlockSpec((1, d), lambda i: (0, 0)),
            pl.BlockSpec((d, d), lambda i: (0, 0)),
            pl.BlockSpec((1, d), lambda i: (0, 0)),
            pl.BlockSpec((p, d), lambda i: (0, 0)),
            pl.BlockSpec((1, p), lambda i: (0, 0)),
            pl.BlockSpec((q, 2 * q), lambda i: (0, 0)),
            pl.BlockSpec((1, q), lambda i: (0, 0)),
        ],
        out_specs=pl.BlockSpec((block_rows, d), lambda i: (i, 0)),
        out_shape=jax.ShapeDtypeStruct((n_nodes, d), jnp.float32),
    )


def kernel(t, u, edge_index, intensity, WF, bF, WG, bG, WZ, bZ, WA, bA):
    n, d = u.shape
    q = intensity.shape[1]
    p = d - q
    e = edge_index.shape[1]
    w = q + 8

    src = edge_index[0].astype(jnp.int32)
    dst = edge_index[1].astype(jnp.int32)

    per_w = CHUNK * NW
    e_pad = (e + per_w - 1) // per_w * per_w
    n_chunks = e_pad // per_w
    src_p = jnp.concatenate(
        [src, jnp.zeros((e_pad - e,), jnp.int32)]).reshape(NW, n_chunks, CHUNK)
    dst_p = jnp.concatenate(
        [dst, jnp.full((e_pad - e,), n, jnp.int32)]).reshape(NW, n_chunks, CHUNK)

    acc_rows = ((n + NS) + NS * 8 - 1) // (NS * 8) * 8 * NS

    table = jnp.concatenate(
        [intensity, jnp.ones((n, 1), jnp.float32),
         jnp.zeros((n, w - q - 1), jnp.float32)], axis=1)
    table = jnp.concatenate(
        [table, jnp.zeros((acc_rows - n, w), jnp.float32)], axis=0)

    zeros = jnp.zeros((acc_rows, w), jnp.float32)

    agg = _make_agg(n, n_chunks, w)
    partials = agg(src_p, dst_p, table, zeros)

    block_rows = 1000
    dense = _make_dense(n, d, p, q, block_rows)
    return dense(u, intensity, partials, WF, bF.reshape(1, d), WG,
                 bG.reshape(1, d), WZ, bZ.reshape(1, p), WA,
                 bA.reshape(1, q))


# speedup vs baseline: 1.0156x; 1.0156x over previous
import functools

import jax
import jax.numpy as jnp
from jax import lax
from jax.experimental import pallas as pl
from jax.experimental.pallas import tpu as pltpu
from jax.experimental.pallas import tpu_sc as plsc

NC = 2
NS = 16
NW = NC * NS
CHUNK = 320


def _make_agg(n_nodes, n_chunks, width):
    rows_per_tile = ((n_nodes + NS) + NS * 8 - 1) // (NS * 8) * 8
    acc_rows = rows_per_tile * NS

    mesh = plsc.VectorSubcoreMesh(
        core_axis_name="c", subcore_axis_name="s", num_cores=NC,
        num_subcores=NS)

    @functools.partial(
        pl.kernel,
        out_type=jax.ShapeDtypeStruct((NC, acc_rows, width), jnp.float32),
        mesh=mesh,
        scratch_types=[
            pltpu.VMEM((n_chunks, CHUNK), jnp.int32),
            pltpu.VMEM((n_chunks, CHUNK), jnp.int32),
            pltpu.VMEM((CHUNK, width), jnp.float32),
            pltpu.VMEM_SHARED((acc_rows, width), jnp.float32),
            pltpu.VMEM_SHARED((acc_rows, width), jnp.float32),
            pltpu.SemaphoreType.DMA,
        ],
        compiler_params=pltpu.CompilerParams(use_tc_tiling_on_sc=False),
    )
    def agg(src_hbm, dst_hbm, table_hbm, zeros_hbm, out_hbm,
            src_v, dst_v, rows_a, acc_sh, table_sh, sem):
        c = lax.axis_index("c")
        s = lax.axis_index("s")
        wid = c * NS + s

        z0 = s * rows_per_tile
        pltpu.sync_copy(zeros_hbm.at[pl.ds(z0, rows_per_tile)],
                        acc_sh.at[pl.ds(z0, rows_per_tile)])
        pltpu.sync_copy(table_hbm.at[pl.ds(z0, rows_per_tile)],
                        table_sh.at[pl.ds(z0, rows_per_tile)])
        pltpu.sync_copy(src_hbm.at[wid], src_v)
        pltpu.sync_copy(dst_hbm.at[wid], dst_v)
        plsc.subcore_barrier()

        def chunk_body(j, carry):
            pltpu.async_copy(table_sh.at[src_v.at[j]], rows_a, sem).wait()
            pltpu.sync_copy(rows_a, acc_sh.at[dst_v.at[j]], add=True)
            return carry

        lax.fori_loop(0, n_chunks, chunk_body, 0)
        plsc.subcore_barrier()

        pltpu.sync_copy(acc_sh.at[pl.ds(z0, rows_per_tile)],
                        out_hbm.at[c, pl.ds(z0, rows_per_tile)])

    return agg


def _make_dense(n_nodes, d, p, q, block_rows):
    grid = (n_nodes // block_rows,)
    w = q + 8

    def body(u_ref, it_ref, ps_ref, wf_ref, bf_ref, wg_ref, bg_ref,
             wz_ref, bz_ref, wa_ref, ba_ref, out_ref):
        x = u_ref[...]
        dn = (((1,), (1,)), ((), ()))
        fu = jax.nn.softplus(
            lax.dot_general(x, wf_ref[...], dn,
                            preferred_element_type=jnp.float32) + bf_ref[...])
        gu = jax.nn.softplus(
            lax.dot_general(x, wg_ref[...], dn,
                            preferred_element_type=jnp.float32) + bg_ref[...])
        zu = jnp.tanh(
            lax.dot_general(x, wz_ref[...], dn,
                            preferred_element_type=jnp.float32) + bz_ref[...])
        ps = ps_ref[0] + ps_ref[1]
        nbr_sum = ps[:, :q]
        deg = ps[:, q:q + 1]
        nbr_mean = jnp.where(deg > 0, nbr_sum / jnp.maximum(deg, 1.0), 0.0)
        agi = jnp.concatenate([it_ref[...], nbr_mean], axis=1)
        ag = jnp.maximum(
            lax.dot_general(agi, wa_ref[...], dn,
                            preferred_element_type=jnp.float32) + ba_ref[...],
            0.0)
        du = -fu * x + gu * jnp.concatenate([zu, ag], axis=1)
        up = x[:, :p]
        dup = du[:, :p]
        s1 = jnp.sum(dup * up, axis=1, keepdims=True)
        s2 = jnp.sum(up * up, axis=1, keepdims=True)
        dup = dup - (s1 / s2) * up
        out_ref[...] = jnp.concatenate([dup, du[:, p:]], axis=1)

    return pl.pallas_call(
        body,
        grid=grid,
        in_specs=[
            pl.BlockSpec((block_rows, d), lambda i: (i, 0)),
            pl.BlockSpec((block_rows, q), lambda i: (i, 0)),
            pl.BlockSpec((NC, block_rows, w), lambda i: (0, i, 0)),
            pl.BlockSpec((d, d), lambda i: (0, 0)),
            pl.BlockSpec((1, d), lambda i: (0, 0)),
            pl.BlockSpec((d, d), lambda i: (0, 0)),
            pl.BlockSpec((1, d), lambda i: (0, 0)),
            pl.BlockSpec((p, d), lambda i: (0, 0)),
            pl.BlockSpec((1, p), lambda i: (0, 0)),
            pl.BlockSpec((q, 2 * q), lambda i: (0, 0)),
            pl.BlockSpec((1, q), lambda i: (0, 0)),
        ],
        out_specs=pl.BlockSpec((block_rows, d), lambda i: (i, 0)),
        out_shape=jax.ShapeDtypeStruct((n_nodes, d), jnp.float32),
    )


def kernel(t, u, edge_index, intensity, WF, bF, WG, bG, WZ, bZ, WA, bA):
    n, d = u.shape
    q = intensity.shape[1]
    p = d - q
    e = edge_index.shape[1]
    w = q + 8

    src = edge_index[0].astype(jnp.int32)
    dst = edge_index[1].astype(jnp.int32)

    per_w = CHUNK * NW
    e_pad = (e + per_w - 1) // per_w * per_w
    n_chunks = e_pad // per_w
    src_p = jnp.concatenate(
        [src, jnp.zeros((e_pad - e,), jnp.int32)]).reshape(NW, n_chunks, CHUNK)
    dst_p = jnp.concatenate(
        [dst, jnp.full((e_pad - e,), n, jnp.int32)]).reshape(NW, n_chunks, CHUNK)

    acc_rows = ((n + NS) + NS * 8 - 1) // (NS * 8) * 8 * NS

    table = jnp.concatenate(
        [intensity, jnp.ones((n, 1), jnp.float32),
         jnp.zeros((n, w - q - 1), jnp.float32)], axis=1)
    table = jnp.concatenate(
        [table, jnp.zeros((acc_rows - n, w), jnp.float32)], axis=0)

    zeros = jnp.zeros((acc_rows, w), jnp.float32)

    agg = _make_agg(n, n_chunks, w)
    partials = agg(src_p, dst_p, table, zeros)

    block_rows = 1000
    dense = _make_dense(n, d, p, q, block_rows)
    return dense(u, intensity, partials, WF, bF.reshape(1, d), WG,
                 bG.reshape(1, d), WZ, bZ.reshape(1, p), WA,
                 bA.reshape(1, q))


# speedup vs baseline: 1.0383x; 1.0224x over previous
import functools

import jax
import jax.numpy as jnp
from jax import lax
from jax.experimental import pallas as pl
from jax.experimental.pallas import tpu as pltpu
from jax.experimental.pallas import tpu_sc as plsc

NC = 2
NS = 16
NW = NC * NS
CHUNK = 320


def _make_agg(n_nodes, n_chunks, width):
    rows_per_tile = ((n_nodes + NS) + NS * 8 - 1) // (NS * 8) * 8
    acc_rows = rows_per_tile * NS

    mesh = plsc.VectorSubcoreMesh(
        core_axis_name="c", subcore_axis_name="s", num_cores=NC,
        num_subcores=NS)

    @functools.partial(
        pl.kernel,
        out_type=jax.ShapeDtypeStruct((NC, acc_rows, width), jnp.float32),
        mesh=mesh,
        scratch_types=[
            pltpu.VMEM((n_chunks, CHUNK), jnp.int32),
            pltpu.VMEM((n_chunks, CHUNK), jnp.int32),
            pltpu.VMEM((CHUNK, width), jnp.float32),
            pltpu.VMEM_SHARED((acc_rows, width), jnp.float32),
            pltpu.VMEM_SHARED((acc_rows, width), jnp.float32),
            pltpu.SemaphoreType.DMA,
        ],
        compiler_params=pltpu.CompilerParams(use_tc_tiling_on_sc=False),
    )
    def agg(src_hbm, dst_hbm, table_hbm, zeros_hbm, out_hbm,
            src_v, dst_v, rows_a, acc_sh, table_sh, sem):
        c = lax.axis_index("c")
        s = lax.axis_index("s")
        wid = c * NS + s

        z0 = s * rows_per_tile
        pltpu.sync_copy(zeros_hbm.at[pl.ds(z0, rows_per_tile)],
                        acc_sh.at[pl.ds(z0, rows_per_tile)])
        pltpu.sync_copy(table_hbm.at[pl.ds(z0, rows_per_tile)],
                        table_sh.at[pl.ds(z0, rows_per_tile)])
        pltpu.sync_copy(src_hbm.at[wid], src_v)
        pltpu.sync_copy(dst_hbm.at[wid], dst_v)
        plsc.subcore_barrier()

        def chunk_body(j, carry):
            pltpu.async_copy(table_sh.at[src_v.at[j]], rows_a, sem).wait()
            pltpu.sync_copy(rows_a, acc_sh.at[dst_v.at[j]], add=True)
            return carry

        lax.fori_loop(0, n_chunks, chunk_body, 0)
        plsc.subcore_barrier()

        pltpu.sync_copy(acc_sh.at[pl.ds(z0, rows_per_tile)],
                        out_hbm.at[c, pl.ds(z0, rows_per_tile)])

    return agg


def _make_dense(n_nodes, d, p, q, block_rows):
    grid = (n_nodes // block_rows,)
    w = q + 8

    def body(u_ref, it_ref, ps_ref, wf_ref, bf_ref, wg_ref, bg_ref,
             wz_ref, bz_ref, wa_ref, ba_ref, out_ref):
        x = u_ref[...]
        dn = (((1,), (1,)), ((), ()))
        fu = jax.nn.softplus(
            lax.dot_general(x, wf_ref[...], dn,
                            preferred_element_type=jnp.float32) + bf_ref[...])
        gu = jax.nn.softplus(
            lax.dot_general(x, wg_ref[...], dn,
                            preferred_element_type=jnp.float32) + bg_ref[...])
        zu = jnp.tanh(
            lax.dot_general(x, wz_ref[...], dn,
                            preferred_element_type=jnp.float32) + bz_ref[...])
        ps = ps_ref[0] + ps_ref[1]
        nbr_sum = ps[:, :q]
        deg = ps[:, q:q + 1]
        nbr_mean = jnp.where(deg > 0, nbr_sum / jnp.maximum(deg, 1.0), 0.0)
        agi = jnp.concatenate([it_ref[...], nbr_mean], axis=1)
        ag = jnp.maximum(
            lax.dot_general(agi, wa_ref[...], dn,
                            preferred_element_type=jnp.float32) + ba_ref[...],
            0.0)
        du = -fu * x + gu * jnp.concatenate([zu, ag], axis=1)
        up = x[:, :p]
        dup = du[:, :p]
        s1 = jnp.sum(dup * up, axis=1, keepdims=True)
        s2 = jnp.sum(up * up, axis=1, keepdims=True)
        dup = dup - (s1 / s2) * up
        out_ref[...] = jnp.concatenate([dup, du[:, p:]], axis=1)

    return pl.pallas_call(
        body,
        grid=grid,
        in_specs=[
            pl.BlockSpec((block_rows, d), lambda i: (i, 0)),
            pl.BlockSpec((block_rows, q), lambda i: (i, 0)),
            pl.BlockSpec((NC, block_rows, w), lambda i: (0, i, 0)),
            pl.BlockSpec((d, d), lambda i: (0, 0)),
            pl.BlockSpec((1, d), lambda i: (0, 0)),
            pl.BlockSpec((d, d), lambda i: (0, 0)),
            pl.BlockSpec((1, d), lambda i: (0, 0)),
            pl.BlockSpec((p, d), lambda i: (0, 0)),
            pl.BlockSpec((1, p), lambda i: (0, 0)),
            pl.BlockSpec((q, 2 * q), lambda i: (0, 0)),
            pl.BlockSpec((1, q), lambda i: (0, 0)),
        ],
        out_specs=pl.BlockSpec((block_rows, d), lambda i: (i, 0)),
        out_shape=jax.ShapeDtypeStruct((n_nodes, d), jnp.float32),
    )


def kernel(t, u, edge_index, intensity, WF, bF, WG, bG, WZ, bZ, WA, bA):
    n, d = u.shape
    q = intensity.shape[1]
    p = d - q
    e = edge_index.shape[1]
    w = q + 8

    src = edge_index[0].astype(jnp.int32)
    dst = edge_index[1].astype(jnp.int32)

    per_w = CHUNK * NW
    e_pad = (e + per_w - 1) // per_w * per_w
    n_chunks = e_pad // per_w
    src_p = jnp.concatenate(
        [src, jnp.zeros((e_pad - e,), jnp.int32)]).reshape(NW, n_chunks, CHUNK)
    dst_p = jnp.concatenate(
        [dst, jnp.full((e_pad - e,), n, jnp.int32)]).reshape(NW, n_chunks, CHUNK)

    acc_rows = ((n + NS) + NS * 8 - 1) // (NS * 8) * 8 * NS

    table = jnp.concatenate(
        [intensity, jnp.ones((n, 1), jnp.float32),
         jnp.zeros((n, w - q - 1), jnp.float32)], axis=1)
    table = jnp.concatenate(
        [table, jnp.zeros((acc_rows - n, w), jnp.float32)], axis=0)

    zeros = jnp.zeros((acc_rows, w), jnp.float32)

    agg = _make_agg(n, n_chunks, w)
    partials = agg(src_p, dst_p, table, zeros)

    block_rows = 2000
    dense = _make_dense(n, d, p, q, block_rows)
    return dense(u, intensity, partials, WF, bF.reshape(1, d), WG,
                 bG.reshape(1, d), WZ, bZ.reshape(1, p), WA,
                 bA.reshape(1, q))


# speedup vs baseline: 1.1410x; 1.0989x over previous
import functools

import jax
import jax.numpy as jnp
from jax import lax
from jax.experimental import pallas as pl
from jax.experimental.pallas import tpu as pltpu
from jax.experimental.pallas import tpu_sc as plsc

NC = 2
NS = 16
NW = NC * NS
CHUNK = 320


def _make_agg(n_nodes, n_chunks, width):
    rows_per_tile = ((n_nodes + NS) + NS * 8 - 1) // (NS * 8) * 8
    acc_rows = rows_per_tile * NS

    mesh = plsc.VectorSubcoreMesh(
        core_axis_name="c", subcore_axis_name="s", num_cores=NC,
        num_subcores=NS)

    @functools.partial(
        pl.kernel,
        out_type=jax.ShapeDtypeStruct((NC, acc_rows, 128), jnp.float32),
        mesh=mesh,
        scratch_types=[
            pltpu.VMEM((n_chunks, CHUNK), jnp.int32),
            pltpu.VMEM((n_chunks, CHUNK), jnp.int32),
            pltpu.VMEM((CHUNK, width), jnp.float32),
            pltpu.VMEM_SHARED((acc_rows, width), jnp.float32),
            pltpu.VMEM_SHARED((acc_rows, width), jnp.float32),
            pltpu.SemaphoreType.DMA,
        ],
        compiler_params=pltpu.CompilerParams(use_tc_tiling_on_sc=False),
    )
    def agg(src_hbm, dst_hbm, table_hbm, zeros_hbm, out_hbm,
            src_v, dst_v, rows_a, acc_sh, table_sh, sem):
        c = lax.axis_index("c")
        s = lax.axis_index("s")
        wid = c * NS + s

        z0 = s * rows_per_tile
        pltpu.sync_copy(zeros_hbm.at[pl.ds(z0, rows_per_tile)],
                        acc_sh.at[pl.ds(z0, rows_per_tile)])
        pltpu.sync_copy(table_hbm.at[pl.ds(z0, rows_per_tile)],
                        table_sh.at[pl.ds(z0, rows_per_tile)])
        pltpu.sync_copy(src_hbm.at[wid], src_v)
        pltpu.sync_copy(dst_hbm.at[wid], dst_v)
        plsc.subcore_barrier()

        def chunk_body(j, carry):
            pltpu.async_copy(table_sh.at[src_v.at[j]], rows_a, sem).wait()
            pltpu.sync_copy(rows_a, acc_sh.at[dst_v.at[j]], add=True)
            return carry

        lax.fori_loop(0, n_chunks, chunk_body, 0)
        plsc.subcore_barrier()

        pltpu.sync_copy(
            acc_sh.at[pl.ds(z0, rows_per_tile)],
            out_hbm.at[c, pl.ds(z0, rows_per_tile), pl.ds(0, width)])

    return agg


def _make_dense(n_nodes, d, p, q, block_rows):
    grid = (n_nodes // block_rows,)
    w = q + 8

    def body(u_ref, it_ref, ps_ref, wf_ref, bf_ref, wg_ref, bg_ref,
             wz_ref, bz_ref, wa_ref, ba_ref, out_ref):
        x = u_ref[...]
        dn = (((1,), (1,)), ((), ()))
        fu = jax.nn.softplus(
            lax.dot_general(x, wf_ref[...], dn,
                            preferred_element_type=jnp.float32) + bf_ref[...])
        gu = jax.nn.softplus(
            lax.dot_general(x, wg_ref[...], dn,
                            preferred_element_type=jnp.float32) + bg_ref[...])
        zu = jnp.tanh(
            lax.dot_general(x, wz_ref[...], dn,
                            preferred_element_type=jnp.float32) + bz_ref[...])
        ps = ps_ref[0] + ps_ref[1]
        nbr_sum = ps[:, :q]
        deg = ps[:, q:q + 1]
        nbr_mean = jnp.where(deg > 0, nbr_sum / jnp.maximum(deg, 1.0), 0.0)
        agi = jnp.concatenate([it_ref[...], nbr_mean], axis=1)
        ag = jnp.maximum(
            lax.dot_general(agi, wa_ref[...], dn,
                            preferred_element_type=jnp.float32) + ba_ref[...],
            0.0)
        du = -fu * x + gu * jnp.concatenate([zu, ag], axis=1)
        up = x[:, :p]
        dup = du[:, :p]
        s1 = jnp.sum(dup * up, axis=1, keepdims=True)
        s2 = jnp.sum(up * up, axis=1, keepdims=True)
        dup = dup - (s1 / s2) * up
        out_ref[...] = jnp.concatenate([dup, du[:, p:]], axis=1)

    return pl.pallas_call(
        body,
        grid=grid,
        in_specs=[
            pl.BlockSpec((block_rows, d), lambda i: (i, 0)),
            pl.BlockSpec((block_rows, q), lambda i: (i, 0)),
            pl.BlockSpec((NC, block_rows, 128), lambda i: (0, i, 0)),
            pl.BlockSpec((d, d), lambda i: (0, 0)),
            pl.BlockSpec((1, d), lambda i: (0, 0)),
            pl.BlockSpec((d, d), lambda i: (0, 0)),
            pl.BlockSpec((1, d), lambda i: (0, 0)),
            pl.BlockSpec((p, d), lambda i: (0, 0)),
            pl.BlockSpec((1, p), lambda i: (0, 0)),
            pl.BlockSpec((q, 2 * q), lambda i: (0, 0)),
            pl.BlockSpec((1, q), lambda i: (0, 0)),
        ],
        out_specs=pl.BlockSpec((block_rows, d), lambda i: (i, 0)),
        out_shape=jax.ShapeDtypeStruct((n_nodes, d), jnp.float32),
    )


def kernel(t, u, edge_index, intensity, WF, bF, WG, bG, WZ, bZ, WA, bA):
    n, d = u.shape
    q = intensity.shape[1]
    p = d - q
    e = edge_index.shape[1]
    w = q + 8

    src = edge_index[0].astype(jnp.int32)
    dst = edge_index[1].astype(jnp.int32)

    per_w = CHUNK * NW
    e_pad = (e + per_w - 1) // per_w * per_w
    n_chunks = e_pad // per_w
    src_p = jnp.concatenate(
        [src, jnp.zeros((e_pad - e,), jnp.int32)]).reshape(NW, n_chunks, CHUNK)
    dst_p = jnp.concatenate(
        [dst, jnp.full((e_pad - e,), n, jnp.int32)]).reshape(NW, n_chunks, CHUNK)

    acc_rows = ((n + NS) + NS * 8 - 1) // (NS * 8) * 8 * NS

    table = jnp.concatenate(
        [intensity, jnp.ones((n, 1), jnp.float32),
         jnp.zeros((n, w - q - 1), jnp.float32)], axis=1)
    table = jnp.concatenate(
        [table, jnp.zeros((acc_rows - n, w), jnp.float32)], axis=0)

    zeros = jnp.zeros((acc_rows, w), jnp.float32)

    agg = _make_agg(n, n_chunks, w)
    partials = agg(src_p, dst_p, table, zeros)

    block_rows = 2000
    dense = _make_dense(n, d, p, q, block_rows)
    return dense(u, intensity, partials, WF, bF.reshape(1, d), WG,
                 bG.reshape(1, d), WZ, bZ.reshape(1, p), WA,
                 bA.reshape(1, q))
